# baseline (speedup 1.0000x reference)
import jax
import jax.numpy as jnp
from jax import lax
from jax.experimental import pallas as pl
from jax.experimental.pallas import tpu as pltpu
from jax.experimental.pallas import tpu_sc as plsc

N_ROWS = 320000
N_SEG = 10000
D_IN = 128
D_HID = 128
N_OUT = 2

LANES = 128
NUM_CORES = 2
NUM_SUBCORES = 16
NUM_WORKERS = NUM_CORES * NUM_SUBCORES
SEG_CHUNK = 1000
ZROWS = N_SEG // NUM_SUBCORES

C_CHUNKS = 2
CH_ROWS = N_ROWS // C_CHUNKS
CH_GROUPS = CH_ROWS // LANES
CG_BASE = CH_GROUPS // NUM_WORKERS
CG_REM = CH_GROUPS % NUM_WORKERS

MM_BLOCK = 6400
TAIL_BLOCK = 2000


def _mm_body(x_ref, w_ref, b_ref, o_ref):
    acc = jnp.dot(x_ref[...], w_ref[...], preferred_element_type=jnp.float32)
    o_ref[...] = jnp.maximum(acc + b_ref[...], 0.0)


def _make_seg_body(chunk):
    def _seg_body(h_hbm, seg_hbm, z2_hbm, z1_hbm, sums_hbm, cnts_hbm,
                  rows0_v, rows1_v, idx_v, ones_v, z1_v,
                  semr0, semr1, semi0, semi1, sums_sh, cnts_sh):
        c = lax.axis_index("c")
        s = lax.axis_index("s")
        w = s * NUM_CORES + c
        slots = ((rows0_v, semr0, semi0), (rows1_v, semr1, semi1))

        for j in range(LANES // 16):
            ones_v[pl.ds(j * 16, 16)] = jnp.ones((16,), jnp.float32)

        pltpu.sync_copy(z2_hbm, sums_sh.at[pl.ds(s * ZROWS, ZROWS)])

        @pl.when(s < N_SEG // SEG_CHUNK)
        def _():
            pltpu.sync_copy(z1_hbm, z1_v)
            pltpu.sync_copy(z1_v.at[pl.ds(0, SEG_CHUNK)],
                            cnts_sh.at[pl.ds(s * SEG_CHUNK, SEG_CHUNK)])

        plsc.subcore_barrier()

        ng = CG_BASE + jnp.where(w < CG_REM, 1, 0)
        base = chunk * CH_GROUPS + w * CG_BASE + jnp.minimum(w, CG_REM)

        def start(b, i):
            rows_b, semr_b, semi_b = slots[b]
            g = base + i
            pltpu.async_copy(h_hbm.at[pl.ds((g - chunk * CH_GROUPS) * LANES,
                                            LANES)], rows_b, semr_b)
            pltpu.async_copy(seg_hbm.at[pl.ds(g * LANES, LANES)],
                             idx_v.at[b], semi_b)

        def wait(b):
            rows_b, semr_b, semi_b = slots[b]
            pltpu.make_async_copy(h_hbm.at[pl.ds(0, LANES)], rows_b,
                                  semr_b).wait()
            pltpu.make_async_copy(seg_hbm.at[pl.ds(0, LANES)], idx_v.at[b],
                                  semi_b).wait()

        start(0, 0)

        @pl.when(ng > 1)
        def _():
            start(1, 1)

        def body(i2, carry):
            for b in range(2):
                i = i2 * 2 + b

                @pl.when(i < ng)
                def _():
                    rows_b, _, _ = slots[b]
                    wait(b)
                    idx = idx_v.at[b]
                    pltpu.sync_copy(rows_b, sums_sh.at[idx], add=True)
                    pltpu.sync_copy(ones_v, cnts_sh.at[idx], add=True)

                    @pl.when(i + 2 < ng)
                    def _():
                        start(b, i + 2)

            return carry

        lax.fori_loop(0, (CG_BASE + 2) // 2, body, 0)
        plsc.subcore_barrier()

        @pl.when(s < N_SEG // SEG_CHUNK)
        def _():
            sl = pl.ds(s * SEG_CHUNK, SEG_CHUNK)
            pltpu.sync_copy(sums_sh.at[sl], sums_hbm.at[c, sl])
            pltpu.sync_copy(cnts_sh.at[sl], z1_v.at[pl.ds(0, SEG_CHUNK)])
            pltpu.sync_copy(
                z1_v.at[pl.ds(0, SEG_CHUNK)],
                cnts_hbm.at[pl.ds(c * N_SEG + s * SEG_CHUNK, SEG_CHUNK)])

    return _seg_body


def _tail_body(*refs):
    sums_refs = refs[:C_CHUNKS]
    cnt_refs = refs[C_CHUNKS:2 * C_CHUNKS]
    w2_ref, b2_ref, w3_ref, b3_ref, w4_ref, b4_ref = refs[2 * C_CHUNKS:-2]
    logits_ref, probs_ref = refs[-2:]

    s = sums_refs[0][0] + sums_refs[0][1]
    cnt = cnt_refs[0][:, 0] + cnt_refs[0][:, 1]
    for k in range(1, C_CHUNKS):
        s = s + sums_refs[k][0] + sums_refs[k][1]
        cnt = cnt + cnt_refs[k][:, 0] + cnt_refs[k][:, 1]

    x = jnp.dot(s, w2_ref[...], preferred_element_type=jnp.float32)
    x = x / jnp.maximum(cnt, 1.0)[:, None]
    x = x + b2_ref[...] * jnp.where(cnt > 0.0, 1.0, 0.0)[:, None]
    x = jnp.dot(x, w3_ref[...], preferred_element_type=jnp.float32)
    x = jnp.maximum(x + b3_ref[...], 0.0)
    l = jnp.dot(x, w4_ref[...], preferred_element_type=jnp.float32)
    l = l + b4_ref[...]
    m = jnp.max(l, axis=-1, keepdims=True)
    e = jnp.exp(l - m)
    p = e / jnp.sum(e, axis=-1, keepdims=True)
    logits_ref[...] = l
    probs_ref[...] = p


def kernel(features, segments, W1, b1, W2, b2, W3, b3, W4, b4):
    f32 = jnp.float32

    zeros2 = jnp.zeros((ZROWS, D_HID), f32)
    zeros1 = jnp.zeros((1008,), f32)

    partial_sums, partial_cnts = [], []
    for k in range(C_CHUNKS):
        h_k = pl.pallas_call(
            _mm_body,
            grid=(CH_ROWS // MM_BLOCK,),
            in_specs=[
                pl.BlockSpec((MM_BLOCK, D_IN),
                             lambda i, k=k: (i + k * (CH_ROWS // MM_BLOCK), 0)),
                pl.BlockSpec((D_IN, D_HID), lambda i: (0, 0)),
                pl.BlockSpec((1, D_HID), lambda i: (0, 0)),
            ],
            out_specs=pl.BlockSpec((MM_BLOCK, D_HID), lambda i: (i, 0)),
            out_shape=jax.ShapeDtypeStruct((CH_ROWS, D_HID), f32),
        )(features, W1, b1.reshape(1, D_HID))

        s_k, c_k = pl.kernel(
            _make_seg_body(k),
            out_type=(
                jax.ShapeDtypeStruct((NUM_CORES, N_SEG, D_HID), f32),
                jax.ShapeDtypeStruct((NUM_CORES * N_SEG,), f32),
            ),
            mesh=plsc.VectorSubcoreMesh(core_axis_name="c",
                                        subcore_axis_name="s"),
            scratch_types=[
                pltpu.VMEM((LANES, D_HID), f32),
                pltpu.VMEM((LANES, D_HID), f32),
                pltpu.VMEM((2, LANES), jnp.int32),
                pltpu.VMEM((LANES,), f32),
                pltpu.VMEM((1008,), f32),
                pltpu.SemaphoreType.DMA,
                pltpu.SemaphoreType.DMA,
                pltpu.SemaphoreType.DMA,
                pltpu.SemaphoreType.DMA,
                pltpu.VMEM_SHARED((N_SEG, D_HID), f32),
                pltpu.VMEM_SHARED((N_SEG,), f32),
            ],
        )(h_k, segments, zeros2, zeros1)
        partial_sums.append(s_k)
        partial_cnts.append(c_k.reshape(NUM_CORES, N_SEG).T)

    sums_specs = [pl.BlockSpec((NUM_CORES, TAIL_BLOCK, D_HID),
                               lambda i: (0, i, 0)) for _ in range(C_CHUNKS)]
    cnt_specs = [pl.BlockSpec((TAIL_BLOCK, NUM_CORES), lambda i: (i, 0))
                 for _ in range(C_CHUNKS)]

    logits, probs = pl.pallas_call(
        _tail_body,
        grid=(N_SEG // TAIL_BLOCK,),
        in_specs=sums_specs + cnt_specs + [
            pl.BlockSpec((D_HID, D_HID), lambda i: (0, 0)),
            pl.BlockSpec((1, D_HID), lambda i: (0, 0)),
            pl.BlockSpec((D_HID, D_HID), lambda i: (0, 0)),
            pl.BlockSpec((1, D_HID), lambda i: (0, 0)),
            pl.BlockSpec((D_HID, N_OUT), lambda i: (0, 0)),
            pl.BlockSpec((1, N_OUT), lambda i: (0, 0)),
        ],
        out_specs=[
            pl.BlockSpec((TAIL_BLOCK, N_OUT), lambda i: (i, 0)),
            pl.BlockSpec((TAIL_BLOCK, N_OUT), lambda i: (i, 0)),
        ],
        out_shape=[
            jax.ShapeDtypeStruct((N_SEG, N_OUT), f32),
            jax.ShapeDtypeStruct((N_SEG, N_OUT), f32),
        ],
    )(*partial_sums, *partial_cnts, W2, b2.reshape(1, D_HID),
      W3, b3.reshape(1, D_HID), W4, b4.reshape(1, N_OUT))

    return (logits, probs)


# speedup vs baseline: 1.0220x; 1.0220x over previous
import jax
import jax.numpy as jnp
from jax import lax
from jax.experimental import pallas as pl
from jax.experimental.pallas import tpu as pltpu
from jax.experimental.pallas import tpu_sc as plsc

N_ROWS = 320000
N_SEG = 10000
D_IN = 128
D_HID = 128
N_OUT = 2

LANES = 128
NUM_CORES = 2
NUM_SUBCORES = 16
NUM_WORKERS = NUM_CORES * NUM_SUBCORES
SEG_CHUNK = 1000
ZROWS = N_SEG // NUM_SUBCORES

C_CHUNKS = 1
CH_ROWS = N_ROWS // C_CHUNKS
CH_GROUPS = CH_ROWS // LANES
CG_BASE = CH_GROUPS // NUM_WORKERS
CG_REM = CH_GROUPS % NUM_WORKERS

MM_BLOCK = 6400
TAIL_BLOCK = 2000


def _mm_body(x_ref, w_ref, b_ref, o_ref):
    acc = jnp.dot(x_ref[...], w_ref[...], preferred_element_type=jnp.float32)
    o_ref[...] = jnp.maximum(acc + b_ref[...], 0.0)


def _make_seg_body(chunk):
    def _seg_body(h_hbm, seg_hbm, z2_hbm, z1_hbm, sums_hbm, cnts_hbm,
                  data0_v, data1_v, idx_v, ones_v, z1_v,
                  semd0, semd1, semi0, semi1,
                  sums_sh, cnts_sh):
        c = lax.axis_index("c")
        s = lax.axis_index("s")
        w = s * NUM_CORES + c
        data = (data0_v, data1_v)
        semd = (semd0, semd1)
        semi = (semi0, semi1)

        for j in range(LANES // 16):
            ones_v[pl.ds(j * 16, 16)] = jnp.ones((16,), jnp.float32)

        pltpu.sync_copy(z2_hbm, sums_sh.at[pl.ds(s * ZROWS, ZROWS)])

        @pl.when(s < N_SEG // SEG_CHUNK)
        def _():
            pltpu.sync_copy(z1_hbm, z1_v)
            pltpu.sync_copy(z1_v.at[pl.ds(0, SEG_CHUNK)],
                            cnts_sh.at[pl.ds(s * SEG_CHUNK, SEG_CHUNK)])

        plsc.subcore_barrier()

        ng = CG_BASE + jnp.where(w < CG_REM, 1, 0)
        base = chunk * CH_GROUPS + w * CG_BASE + jnp.minimum(w, CG_REM)

        def start_group(slot, i):
            g = base + i
            pltpu.async_copy(
                h_hbm.at[pl.ds((g - chunk * CH_GROUPS) * LANES, LANES)],
                data[slot], semd[slot])
            pltpu.async_copy(seg_hbm.at[pl.ds(g * LANES, LANES)],
                             idx_v.at[slot], semi[slot])

        def wait_group(slot):
            pltpu.make_async_copy(h_hbm.at[pl.ds(0, LANES)],
                                  data[slot], semd[slot]).wait()
            pltpu.make_async_copy(seg_hbm.at[pl.ds(0, LANES)],
                                  idx_v.at[slot], semi[slot]).wait()

        def process(slot):
            idx = idx_v.at[slot]
            pltpu.sync_copy(data[slot], sums_sh.at[idx], add=True)
            pltpu.sync_copy(ones_v, cnts_sh.at[idx], add=True)

        for j in range(2):
            @pl.when(j < ng)
            def _(j=j):
                start_group(j, j)

        def body(i2, carry):
            for j in range(2):
                i = i2 * 2 + j

                @pl.when(i < ng)
                def _(i=i, j=j):
                    wait_group(j)
                    process(j)

                    @pl.when(i + 2 < ng)
                    def _():
                        start_group(j, i + 2)

            return carry

        lax.fori_loop(0, (CG_BASE + 2) // 2, body, 0)

        plsc.subcore_barrier()

        @pl.when(s < N_SEG // SEG_CHUNK)
        def _():
            sl = pl.ds(s * SEG_CHUNK, SEG_CHUNK)
            pltpu.sync_copy(sums_sh.at[sl], sums_hbm.at[c, sl])
            pltpu.sync_copy(cnts_sh.at[sl], z1_v.at[pl.ds(0, SEG_CHUNK)])
            pltpu.sync_copy(
                z1_v.at[pl.ds(0, SEG_CHUNK)],
                cnts_hbm.at[pl.ds(c * N_SEG + s * SEG_CHUNK, SEG_CHUNK)])

    return _seg_body


def _tail_body(*refs):
    sums_refs = refs[:C_CHUNKS]
    cnt_refs = refs[C_CHUNKS:2 * C_CHUNKS]
    w2_ref, b2_ref, w3_ref, b3_ref, w4_ref, b4_ref = refs[2 * C_CHUNKS:-2]
    logits_ref, probs_ref = refs[-2:]

    s = sums_refs[0][0] + sums_refs[0][1]
    cnt = cnt_refs[0][:, 0] + cnt_refs[0][:, 1]
    for k in range(1, C_CHUNKS):
        s = s + sums_refs[k][0] + sums_refs[k][1]
        cnt = cnt + cnt_refs[k][:, 0] + cnt_refs[k][:, 1]

    x = jnp.dot(s, w2_ref[...], preferred_element_type=jnp.float32)
    x = x / jnp.maximum(cnt, 1.0)[:, None]
    x = x + b2_ref[...] * jnp.where(cnt > 0.0, 1.0, 0.0)[:, None]
    x = jnp.dot(x, w3_ref[...], preferred_element_type=jnp.float32)
    x = jnp.maximum(x + b3_ref[...], 0.0)
    l = jnp.dot(x, w4_ref[...], preferred_element_type=jnp.float32)
    l = l + b4_ref[...]
    m = jnp.max(l, axis=-1, keepdims=True)
    e = jnp.exp(l - m)
    p = e / jnp.sum(e, axis=-1, keepdims=True)
    logits_ref[...] = l
    probs_ref[...] = p


def kernel(features, segments, W1, b1, W2, b2, W3, b3, W4, b4):
    f32 = jnp.float32

    zeros2 = jnp.zeros((ZROWS, D_HID), f32)
    zeros1 = jnp.zeros((1008,), f32)

    partial_sums, partial_cnts = [], []
    for k in range(C_CHUNKS):
        h_k = pl.pallas_call(
            _mm_body,
            grid=(CH_ROWS // MM_BLOCK,),
            in_specs=[
                pl.BlockSpec((MM_BLOCK, D_IN),
                             lambda i, k=k: (i + k * (CH_ROWS // MM_BLOCK), 0)),
                pl.BlockSpec((D_IN, D_HID), lambda i: (0, 0)),
                pl.BlockSpec((1, D_HID), lambda i: (0, 0)),
            ],
            out_specs=pl.BlockSpec((MM_BLOCK, D_HID), lambda i: (i, 0)),
            out_shape=jax.ShapeDtypeStruct((CH_ROWS, D_HID), f32),
        )(features, W1, b1.reshape(1, D_HID))

        s_k, c_k = pl.kernel(
            _make_seg_body(k),
            out_type=(
                jax.ShapeDtypeStruct((NUM_CORES, N_SEG, D_HID), f32),
                jax.ShapeDtypeStruct((NUM_CORES * N_SEG,), f32),
            ),
            mesh=plsc.VectorSubcoreMesh(core_axis_name="c",
                                        subcore_axis_name="s"),
            scratch_types=[
                pltpu.VMEM((LANES, D_HID), f32),
                pltpu.VMEM((LANES, D_HID), f32),
                pltpu.VMEM((2, LANES), jnp.int32),
                pltpu.VMEM((LANES,), f32),
                pltpu.VMEM((1008,), f32),
                pltpu.SemaphoreType.DMA,
                pltpu.SemaphoreType.DMA,
                pltpu.SemaphoreType.DMA,
                pltpu.SemaphoreType.DMA,
                pltpu.VMEM_SHARED((N_SEG, D_HID), f32),
                pltpu.VMEM_SHARED((N_SEG,), f32),
            ],
        )(h_k, segments, zeros2, zeros1)
        partial_sums.append(s_k)
        partial_cnts.append(c_k.reshape(NUM_CORES, N_SEG).T)

    sums_specs = [pl.BlockSpec((NUM_CORES, TAIL_BLOCK, D_HID),
                               lambda i: (0, i, 0)) for _ in range(C_CHUNKS)]
    cnt_specs = [pl.BlockSpec((TAIL_BLOCK, NUM_CORES), lambda i: (i, 0))
                 for _ in range(C_CHUNKS)]

    logits, probs = pl.pallas_call(
        _tail_body,
        grid=(N_SEG // TAIL_BLOCK,),
        in_specs=sums_specs + cnt_specs + [
            pl.BlockSpec((D_HID, D_HID), lambda i: (0, 0)),
            pl.BlockSpec((1, D_HID), lambda i: (0, 0)),
            pl.BlockSpec((D_HID, D_HID), lambda i: (0, 0)),
            pl.BlockSpec((1, D_HID), lambda i: (0, 0)),
            pl.BlockSpec((D_HID, N_OUT), lambda i: (0, 0)),
            pl.BlockSpec((1, N_OUT), lambda i: (0, 0)),
        ],
        out_specs=[
            pl.BlockSpec((TAIL_BLOCK, N_OUT), lambda i: (i, 0)),
            pl.BlockSpec((TAIL_BLOCK, N_OUT), lambda i: (i, 0)),
        ],
        out_shape=[
            jax.ShapeDtypeStruct((N_SEG, N_OUT), f32),
            jax.ShapeDtypeStruct((N_SEG, N_OUT), f32),
        ],
    )(*partial_sums, *partial_cnts, W2, b2.reshape(1, D_HID),
      W3, b3.reshape(1, D_HID), W4, b4.reshape(1, N_OUT))

    return (logits, probs)
